# Initial kernel scaffold; baseline (speedup 1.0000x reference)
import jax
import jax.numpy as jnp
from jax.experimental import pallas as pl


def kernel(node_fts, edge_fts, graph_fts, adj_mat, hidden, query_biases, stories_biases, memory_contents, output_biases, w_output_linear, w_final):
    raise NotImplementedError("write your pallas kernel here")


# speedup vs baseline: 6.8856x; 6.8856x over previous
import functools

import numpy as np
import jax
import jax.numpy as jnp
from jax import lax
from jax.experimental import pallas as pl
from jax.experimental.pallas import tpu as pltpu
from jax.experimental.pallas import tpu_sc as plsc

_B = 4
_N = 63
_SENT = 32
_EMB = 16
_MEM = 64
_LOUT = 128
_VOCAB = 100000

_NI = _N + 1
_NSEG = _B * _NI * _MEM
_NQSEG = _B * _NI

_NW = 32
_SEG_PER_W = _NSEG // _NW
_CHUNK_SEGS = 4
_CHUNK_IDX = _CHUNK_SEGS * _SENT
_CHUNKS = _SEG_PER_W // _CHUNK_SEGS
_QSEG_PER_W = _NQSEG // _NW
_QCHUNKS = _QSEG_PER_W // _CHUNK_SEGS

_TV = 2048
_NVB = 49
_VP = _TV * _NVB


def _pos_encoding(sentence_size, embedding_size):
    encoding = np.ones((embedding_size, sentence_size), dtype=np.float32)
    ls = sentence_size + 1
    le = embedding_size + 1
    for i in range(1, le):
        for j in range(1, ls):
            encoding[i - 1, j - 1] = (i - (le - 1) / 2) * (j - (ls - 1) / 2)
    encoding = 1 + 4 * encoding / embedding_size / sentence_size
    return np.transpose(encoding)


_ENC_NP = _pos_encoding(_SENT, _EMB)


def _sc_body(tab_hbm, qtab_hbm, idxs_hbm, idxq_hbm, enc_hbm,
             outs_hbm, outq_hbm,
             idx_v, qidx_v, enc_v, rows_a, rows_b, qrows_a, qrows_b,
             out_v, qout_v, ga, gb, qga, qgb):
    wid = lax.axis_index("c") * 16 + lax.axis_index("s")

    pltpu.sync_copy(idxs_hbm.at[wid], idx_v)
    pltpu.sync_copy(idxq_hbm.at[wid], qidx_v)
    pltpu.sync_copy(enc_hbm, enc_v)

    pltpu.async_copy(qtab_hbm.at[qidx_v.at[0]], qrows_a, qga)
    pltpu.async_copy(qtab_hbm.at[qidx_v.at[1]], qrows_b, qgb)

    def fire(c, rows_ref, sem):
        pltpu.async_copy(tab_hbm.at[idx_v.at[c]], rows_ref, sem)

    def wait(rows_ref, sem):
        pltpu.make_async_copy(tab_hbm.at[idx_v.at[0]], rows_ref, sem).wait()

    def compute(c, rows_ref):
        for k in range(_CHUNK_SEGS):
            am0 = rows_ref[k * _SENT + 0, 0:16] * enc_v[0]
            ao0 = rows_ref[k * _SENT + 0, 16:32] * enc_v[0]
            am1 = rows_ref[k * _SENT + 1, 0:16] * enc_v[1]
            ao1 = rows_ref[k * _SENT + 1, 16:32] * enc_v[1]
            for s in range(2, _SENT, 2):
                am0 = am0 + rows_ref[k * _SENT + s, 0:16] * enc_v[s]
                ao0 = ao0 + rows_ref[k * _SENT + s, 16:32] * enc_v[s]
                am1 = am1 + rows_ref[k * _SENT + s + 1, 0:16] * enc_v[s + 1]
                ao1 = ao1 + rows_ref[k * _SENT + s + 1, 16:32] * enc_v[s + 1]
            seg = c * _CHUNK_SEGS + k
            out_v[seg, 0:16] = am0 + am1
            out_v[seg, 16:32] = ao0 + ao1

    fire(0, rows_a, ga)
    fire(1, rows_b, gb)

    def step(t, carry):
        wait(rows_a, ga)
        compute(2 * t, rows_a)

        @pl.when(t < _CHUNKS // 2 - 1)
        def _():
            fire(2 * t + 2, rows_a, ga)

        wait(rows_b, gb)
        compute(2 * t + 1, rows_b)

        @pl.when(t < _CHUNKS // 2 - 1)
        def _():
            fire(2 * t + 3, rows_b, gb)

        return carry

    lax.fori_loop(0, _CHUNKS // 2, step, 0)

    def qcompute(qrows_ref, kbase):
        for k in range(_CHUNK_SEGS):
            acc0 = qrows_ref[k * _SENT + 0, :] * enc_v[0]
            acc1 = qrows_ref[k * _SENT + 1, :] * enc_v[1]
            for s in range(2, _SENT, 2):
                acc0 = acc0 + qrows_ref[k * _SENT + s, :] * enc_v[s]
                acc1 = acc1 + qrows_ref[k * _SENT + s + 1, :] * enc_v[s + 1]
            qout_v[kbase + k, :] = acc0 + acc1

    pltpu.make_async_copy(qtab_hbm.at[qidx_v.at[0]], qrows_a, qga).wait()
    qcompute(qrows_a, 0)
    pltpu.make_async_copy(qtab_hbm.at[qidx_v.at[1]], qrows_b, qgb).wait()
    qcompute(qrows_b, _CHUNK_SEGS)

    pltpu.sync_copy(out_v, outs_hbm.at[pl.ds(wid * _SEG_PER_W, _SEG_PER_W), :])
    pltpu.sync_copy(qout_v, outq_hbm.at[pl.ds(wid * _QSEG_PER_W, _QSEG_PER_W), :])


@functools.lru_cache(maxsize=None)
def _make_sc_call():
  return functools.partial(
    pl.kernel,
    out_type=[
        jax.ShapeDtypeStruct((_NSEG, 32), jnp.float32),
        jax.ShapeDtypeStruct((_NQSEG, _EMB), jnp.float32),
    ],
    mesh=plsc.VectorSubcoreMesh(core_axis_name="c", subcore_axis_name="s"),
    compiler_params=pltpu.CompilerParams(use_tc_tiling_on_sc=False),
    scratch_types=[
        pltpu.VMEM((_CHUNKS, _CHUNK_IDX), jnp.int32),
        pltpu.VMEM((_QCHUNKS, _CHUNK_IDX), jnp.int32),
        pltpu.VMEM((_SENT, _EMB), jnp.float32),
        pltpu.VMEM((_CHUNK_IDX, 32), jnp.float32),
        pltpu.VMEM((_CHUNK_IDX, 32), jnp.float32),
        pltpu.VMEM((_CHUNK_IDX, _EMB), jnp.float32),
        pltpu.VMEM((_CHUNK_IDX, _EMB), jnp.float32),
        pltpu.VMEM((_SEG_PER_W, 32), jnp.float32),
        pltpu.VMEM((_QSEG_PER_W, _EMB), jnp.float32),
        pltpu.SemaphoreType.DMA,
        pltpu.SemaphoreType.DMA,
        pltpu.SemaphoreType.DMA,
        pltpu.SemaphoreType.DMA,
    ],
  )(_sc_body)


def _tc_body(outs_ref, outq_ref, mb_ref, wout_ref, wfin_ref, o_ref, hf_scr):
    j = pl.program_id(0)

    @pl.when(j == 0)
    def _():
        mem = outs_ref[:, :, 0:16] + mb_ref[...][None, :, :]
        q = outq_ref[...]
        logits = jnp.sum(mem * q[:, None, :], axis=2)
        m = jnp.max(logits, axis=1, keepdims=True)
        e = jnp.exp(logits - m)
        probs = e / jnp.sum(e, axis=1, keepdims=True)
        outr = outs_ref[:, :, 16:32]
        lay = jnp.sum(outr * probs[:, :, None], axis=1)
        a = q + lay
        act = jnp.maximum(
            jnp.dot(a, wout_ref[...], preferred_element_type=jnp.float32), 0.0)
        r = act.reshape(_B, _NI, _LOUT)
        hf = r + r[:, _NI - 1:_NI, :]
        hf_scr[...] = hf.reshape(_B * _NI, _LOUT)

    o_ref[...] = jnp.dot(hf_scr[...], wfin_ref[...],
                         preferred_element_type=jnp.float32)


_tc_call = pl.pallas_call(
    _tc_body,
    grid=(_NVB,),
    in_specs=[
        pl.BlockSpec((_NQSEG, _MEM, 32), lambda j: (0, 0, 0)),
        pl.BlockSpec((_NQSEG, _EMB), lambda j: (0, 0)),
        pl.BlockSpec((_MEM, _EMB), lambda j: (0, 0)),
        pl.BlockSpec((_EMB, _LOUT), lambda j: (0, 0)),
        pl.BlockSpec((_LOUT, _TV), lambda j: (0, j)),
    ],
    out_specs=pl.BlockSpec((_NQSEG, _TV), lambda j: (0, j)),
    out_shape=jax.ShapeDtypeStruct((_NQSEG, _VP), jnp.float32),
    scratch_shapes=[pltpu.VMEM((_NQSEG, _LOUT), jnp.float32)],
)


def kernel(node_fts, edge_fts, graph_fts, adj_mat, hidden,
           query_biases, stories_biases, memory_contents, output_biases,
           w_output_linear, w_final):
    del hidden
    f32 = jnp.float32
    nil = jnp.zeros((1, _EMB), f32)
    tab = jnp.concatenate(
        [jnp.concatenate([stories_biases, nil], axis=0),
         jnp.concatenate([output_biases, nil], axis=0)], axis=1)
    qtab = jnp.concatenate([query_biases, nil], axis=0)

    ef = edge_fts * adj_mat[..., None]
    ef = jnp.pad(ef, ((0, 0), (0, 1), (0, 1), (0, 0)))
    idx_s = ef.astype(jnp.int32).reshape(_NW, _CHUNKS, _CHUNK_IDX)
    qf = jnp.concatenate([node_fts, graph_fts[:, None]], axis=1)
    idx_q = qf.astype(jnp.int32).reshape(_NW, _QCHUNKS, _CHUNK_IDX)

    enc = jnp.asarray(_ENC_NP)

    outs, outq = _make_sc_call()(tab, qtab, idx_s, idx_q, enc)
    outs = outs.reshape(_NQSEG, _MEM, 32)

    wfin_p = jnp.pad(w_final, ((0, 0), (0, _VP - _VOCAB)))
    full = _tc_call(outs, outq, memory_contents, w_output_linear, wfin_p)
    return full.reshape(_B, _NI, _VP)[:, :_N, :_VOCAB]


# speedup vs baseline: 6.9860x; 1.0146x over previous
import functools

import numpy as np
import jax
import jax.numpy as jnp
from jax import lax
from jax.experimental import pallas as pl
from jax.experimental.pallas import tpu as pltpu
from jax.experimental.pallas import tpu_sc as plsc

_B = 4
_N = 63
_SENT = 32
_EMB = 16
_MEM = 64
_LOUT = 128
_VOCAB = 100000

_NI = _N + 1
_NSEG = _B * _NI * _MEM
_NQSEG = _B * _NI

_NW = 32
_SEG_PER_W = _NSEG // _NW
_CHUNK_SEGS = 4
_CHUNK_IDX = _CHUNK_SEGS * _SENT
_CHUNKS = _SEG_PER_W // _CHUNK_SEGS
_QSEG_PER_W = _NQSEG // _NW
_QCHUNKS = _QSEG_PER_W // _CHUNK_SEGS

_TV = 2048
_NVB = 49
_VP = _TV * _NVB


def _pos_encoding(sentence_size, embedding_size):
    encoding = np.ones((embedding_size, sentence_size), dtype=np.float32)
    ls = sentence_size + 1
    le = embedding_size + 1
    for i in range(1, le):
        for j in range(1, ls):
            encoding[i - 1, j - 1] = (i - (le - 1) / 2) * (j - (ls - 1) / 2)
    encoding = 1 + 4 * encoding / embedding_size / sentence_size
    return np.transpose(encoding)


_ENC_NP = _pos_encoding(_SENT, _EMB)


def _sc_body(tab_hbm, qtab_hbm, idxs_hbm, idxq_hbm, enc_hbm,
             outs_hbm, outq_hbm,
             idx_v, qidx_v, enc_v, rows_a, rows_b, rows_c, rows_d,
             qrows_a, qrows_b, out_v, qout_v, ga, gb, gc, gd, qga, qgb):
    wid = lax.axis_index("c") * 16 + lax.axis_index("s")

    pltpu.sync_copy(idxs_hbm.at[wid], idx_v)
    pltpu.sync_copy(idxq_hbm.at[wid], qidx_v)
    pltpu.sync_copy(enc_hbm, enc_v)

    pltpu.async_copy(qtab_hbm.at[qidx_v.at[0]], qrows_a, qga)
    pltpu.async_copy(qtab_hbm.at[qidx_v.at[1]], qrows_b, qgb)

    def fire(c, rows_ref, sem):
        pltpu.async_copy(tab_hbm.at[idx_v.at[c]], rows_ref, sem)

    def wait(rows_ref, sem):
        pltpu.make_async_copy(tab_hbm.at[idx_v.at[0]], rows_ref, sem).wait()

    def compute(c, rows_ref):
        am = [None] * _CHUNK_SEGS
        ao = [None] * _CHUNK_SEGS
        for s in range(_SENT):
            e = enc_v[s]
            for k in range(_CHUNK_SEGS):
                rm = rows_ref[k * _SENT + s, 0:16] * e
                ro = rows_ref[k * _SENT + s, 16:32] * e
                am[k] = rm if s == 0 else am[k] + rm
                ao[k] = ro if s == 0 else ao[k] + ro
        for k in range(_CHUNK_SEGS):
            seg = c * _CHUNK_SEGS + k
            out_v[seg, 0:16] = am[k]
            out_v[seg, 16:32] = ao[k]

    bufs = [rows_a, rows_b, rows_c, rows_d]
    sems = [ga, gb, gc, gd]
    for r in range(4):
        fire(r, bufs[r], sems[r])

    def step(t, carry):
        for r in range(4):
            c = 4 * t + r
            wait(bufs[r], sems[r])
            compute(c, bufs[r])

            @pl.when(t < _CHUNKS // 4 - 1)
            def _():
                fire(c + 4, bufs[r], sems[r])

        return carry

    lax.fori_loop(0, _CHUNKS // 4, step, 0)

    def qcompute(qrows_ref, kbase):
        for k in range(_CHUNK_SEGS):
            acc0 = qrows_ref[k * _SENT + 0, :] * enc_v[0]
            acc1 = qrows_ref[k * _SENT + 1, :] * enc_v[1]
            for s in range(2, _SENT, 2):
                acc0 = acc0 + qrows_ref[k * _SENT + s, :] * enc_v[s]
                acc1 = acc1 + qrows_ref[k * _SENT + s + 1, :] * enc_v[s + 1]
            qout_v[kbase + k, :] = acc0 + acc1

    pltpu.make_async_copy(qtab_hbm.at[qidx_v.at[0]], qrows_a, qga).wait()
    qcompute(qrows_a, 0)
    pltpu.make_async_copy(qtab_hbm.at[qidx_v.at[1]], qrows_b, qgb).wait()
    qcompute(qrows_b, _CHUNK_SEGS)

    pltpu.sync_copy(out_v, outs_hbm.at[pl.ds(wid * _SEG_PER_W, _SEG_PER_W), :])
    pltpu.sync_copy(qout_v, outq_hbm.at[pl.ds(wid * _QSEG_PER_W, _QSEG_PER_W), :])


@functools.lru_cache(maxsize=None)
def _make_sc_call():
  return functools.partial(
    pl.kernel,
    out_type=[
        jax.ShapeDtypeStruct((_NSEG, 32), jnp.float32),
        jax.ShapeDtypeStruct((_NQSEG, _EMB), jnp.float32),
    ],
    mesh=plsc.VectorSubcoreMesh(core_axis_name="c", subcore_axis_name="s"),
    compiler_params=pltpu.CompilerParams(use_tc_tiling_on_sc=False),
    scratch_types=[
        pltpu.VMEM((_CHUNKS, _CHUNK_IDX), jnp.int32),
        pltpu.VMEM((_QCHUNKS, _CHUNK_IDX), jnp.int32),
        pltpu.VMEM((_SENT, _EMB), jnp.float32),
        pltpu.VMEM((_CHUNK_IDX, 32), jnp.float32),
        pltpu.VMEM((_CHUNK_IDX, 32), jnp.float32),
        pltpu.VMEM((_CHUNK_IDX, 32), jnp.float32),
        pltpu.VMEM((_CHUNK_IDX, 32), jnp.float32),
        pltpu.VMEM((_CHUNK_IDX, _EMB), jnp.float32),
        pltpu.VMEM((_CHUNK_IDX, _EMB), jnp.float32),
        pltpu.VMEM((_SEG_PER_W, 32), jnp.float32),
        pltpu.VMEM((_QSEG_PER_W, _EMB), jnp.float32),
        pltpu.SemaphoreType.DMA,
        pltpu.SemaphoreType.DMA,
        pltpu.SemaphoreType.DMA,
        pltpu.SemaphoreType.DMA,
        pltpu.SemaphoreType.DMA,
        pltpu.SemaphoreType.DMA,
    ],
  )(_sc_body)


def _tc_body(outs_ref, outq_ref, mb_ref, wout_ref, wfin_ref, o_ref, hf_scr):
    j = pl.program_id(0)

    @pl.when(j == 0)
    def _():
        mem = outs_ref[:, :, 0:16] + mb_ref[...][None, :, :]
        q = outq_ref[...]
        logits = jnp.sum(mem * q[:, None, :], axis=2)
        m = jnp.max(logits, axis=1, keepdims=True)
        e = jnp.exp(logits - m)
        probs = e / jnp.sum(e, axis=1, keepdims=True)
        outr = outs_ref[:, :, 16:32]
        lay = jnp.sum(outr * probs[:, :, None], axis=1)
        a = q + lay
        act = jnp.maximum(
            jnp.dot(a, wout_ref[...], preferred_element_type=jnp.float32), 0.0)
        r = act.reshape(_B, _NI, _LOUT)
        hf = r[:, :_N, :] + r[:, _NI - 1:_NI, :]
        hf_scr[...] = hf.reshape(_B * _N, _LOUT)

    o_ref[...] = jnp.dot(hf_scr[...], wfin_ref[...],
                         preferred_element_type=jnp.float32)


_tc_call = pl.pallas_call(
    _tc_body,
    grid=(_NVB,),
    in_specs=[
        pl.BlockSpec((_NQSEG, _MEM, 32), lambda j: (0, 0, 0)),
        pl.BlockSpec((_NQSEG, _EMB), lambda j: (0, 0)),
        pl.BlockSpec((_MEM, _EMB), lambda j: (0, 0)),
        pl.BlockSpec((_EMB, _LOUT), lambda j: (0, 0)),
        pl.BlockSpec((_LOUT, _TV), lambda j: (0, j)),
    ],
    out_specs=pl.BlockSpec((_B * _N, _TV), lambda j: (0, j)),
    out_shape=jax.ShapeDtypeStruct((_B * _N, _VOCAB), jnp.float32),
    scratch_shapes=[pltpu.VMEM((_B * _N, _LOUT), jnp.float32)],
)


def kernel(node_fts, edge_fts, graph_fts, adj_mat, hidden,
           query_biases, stories_biases, memory_contents, output_biases,
           w_output_linear, w_final):
    del hidden
    f32 = jnp.float32
    nil = jnp.zeros((1, _EMB), f32)
    tab = jnp.concatenate(
        [jnp.concatenate([stories_biases, nil], axis=0),
         jnp.concatenate([output_biases, nil], axis=0)], axis=1)
    qtab = jnp.concatenate([query_biases, nil], axis=0)

    ef = edge_fts * adj_mat[..., None]
    ef = jnp.pad(ef, ((0, 0), (0, 1), (0, 1), (0, 0)))
    idx_s = ef.astype(jnp.int32).reshape(_NW, _CHUNKS, _CHUNK_IDX)
    qf = jnp.concatenate([node_fts, graph_fts[:, None]], axis=1)
    idx_q = qf.astype(jnp.int32).reshape(_NW, _QCHUNKS, _CHUNK_IDX)

    enc = jnp.asarray(_ENC_NP)

    outs, outq = _make_sc_call()(tab, qtab, idx_s, idx_q, enc)
    outs = outs.reshape(_NQSEG, _MEM, 32)

    full = _tc_call(outs, outq, memory_contents, w_output_linear, w_final)
    return full.reshape(_B, _N, _VOCAB)


# speedup vs baseline: 7.1463x; 1.0229x over previous
import functools

import numpy as np
import jax
import jax.numpy as jnp
from jax import lax
from jax.experimental import pallas as pl
from jax.experimental.pallas import tpu as pltpu
from jax.experimental.pallas import tpu_sc as plsc

_B = 4
_N = 63
_SENT = 32
_EMB = 16
_MEM = 64
_LOUT = 128
_VOCAB = 100000

_NI = _N + 1
_NSEG = _B * _NI * _MEM
_NQSEG = _B * _NI

_NW = 32
_SEG_PER_W = _NSEG // _NW
_CHUNK_SEGS = 4
_CHUNK_IDX = _CHUNK_SEGS * _SENT
_CHUNKS = _SEG_PER_W // _CHUNK_SEGS
_QSEG_PER_W = _NQSEG // _NW
_QCHUNKS = _QSEG_PER_W // _CHUNK_SEGS

_TV = 4096
_NVB = 25


def _pos_encoding(sentence_size, embedding_size):
    encoding = np.ones((embedding_size, sentence_size), dtype=np.float32)
    ls = sentence_size + 1
    le = embedding_size + 1
    for i in range(1, le):
        for j in range(1, ls):
            encoding[i - 1, j - 1] = (i - (le - 1) / 2) * (j - (ls - 1) / 2)
    encoding = 1 + 4 * encoding / embedding_size / sentence_size
    return np.transpose(encoding)


_ENC_NP = _pos_encoding(_SENT, _EMB)


_RING = 8


def _sc_body(tab_hbm, qtab_hbm, idxs_hbm, idxq_hbm, enc_hbm,
             outs_hbm, outq_hbm,
             idx_v, qidx_v, enc_v, *rest):
    bufs = list(rest[:_RING])
    qrows_a, qrows_b, out_v, qout_v = rest[_RING:_RING + 4]
    sems = list(rest[_RING + 4:2 * _RING + 4])
    qga, qgb = rest[2 * _RING + 4:]
    wid = lax.axis_index("c") * 16 + lax.axis_index("s")

    pltpu.sync_copy(idxs_hbm.at[wid], idx_v)
    pltpu.sync_copy(idxq_hbm.at[wid], qidx_v)
    pltpu.sync_copy(enc_hbm, enc_v)

    pltpu.async_copy(qtab_hbm.at[qidx_v.at[0]], qrows_a, qga)
    pltpu.async_copy(qtab_hbm.at[qidx_v.at[1]], qrows_b, qgb)

    def fire(c, rows_ref, sem):
        pltpu.async_copy(tab_hbm.at[idx_v.at[c]], rows_ref, sem)

    def wait(rows_ref, sem):
        pltpu.make_async_copy(tab_hbm.at[idx_v.at[0]], rows_ref, sem).wait()

    def compute(c, rows_ref):
        am = [None] * _CHUNK_SEGS
        ao = [None] * _CHUNK_SEGS
        for s in range(_SENT):
            e = enc_v[s]
            for k in range(_CHUNK_SEGS):
                rm = rows_ref[k * _SENT + s, 0:16] * e
                ro = rows_ref[k * _SENT + s, 16:32] * e
                am[k] = rm if s == 0 else am[k] + rm
                ao[k] = ro if s == 0 else ao[k] + ro
        for k in range(_CHUNK_SEGS):
            seg = c * _CHUNK_SEGS + k
            out_v[seg, 0:16] = am[k]
            out_v[seg, 16:32] = ao[k]

    for r in range(_RING):
        fire(r, bufs[r], sems[r])

    def step(t, carry):
        for r in range(_RING):
            c = _RING * t + r
            wait(bufs[r], sems[r])
            compute(c, bufs[r])

            @pl.when(t < _CHUNKS // _RING - 1)
            def _():
                fire(c + _RING, bufs[r], sems[r])

        return carry

    lax.fori_loop(0, _CHUNKS // _RING, step, 0)

    def qcompute(qrows_ref, kbase):
        for k in range(_CHUNK_SEGS):
            acc0 = qrows_ref[k * _SENT + 0, :] * enc_v[0]
            acc1 = qrows_ref[k * _SENT + 1, :] * enc_v[1]
            for s in range(2, _SENT, 2):
                acc0 = acc0 + qrows_ref[k * _SENT + s, :] * enc_v[s]
                acc1 = acc1 + qrows_ref[k * _SENT + s + 1, :] * enc_v[s + 1]
            qout_v[kbase + k, :] = acc0 + acc1

    pltpu.make_async_copy(qtab_hbm.at[qidx_v.at[0]], qrows_a, qga).wait()
    qcompute(qrows_a, 0)
    pltpu.make_async_copy(qtab_hbm.at[qidx_v.at[1]], qrows_b, qgb).wait()
    qcompute(qrows_b, _CHUNK_SEGS)

    pltpu.sync_copy(out_v, outs_hbm.at[pl.ds(wid * _SEG_PER_W, _SEG_PER_W), :])
    pltpu.sync_copy(qout_v, outq_hbm.at[pl.ds(wid * _QSEG_PER_W, _QSEG_PER_W), :])


@functools.lru_cache(maxsize=None)
def _make_sc_call():
  return functools.partial(
    pl.kernel,
    out_type=[
        jax.ShapeDtypeStruct((_NSEG, 32), jnp.float32),
        jax.ShapeDtypeStruct((_NQSEG, _EMB), jnp.float32),
    ],
    mesh=plsc.VectorSubcoreMesh(core_axis_name="c", subcore_axis_name="s"),
    compiler_params=pltpu.CompilerParams(use_tc_tiling_on_sc=False),
    scratch_types=[
        pltpu.VMEM((_CHUNKS, _CHUNK_IDX), jnp.int32),
        pltpu.VMEM((_QCHUNKS, _CHUNK_IDX), jnp.int32),
        pltpu.VMEM((_SENT, _EMB), jnp.float32),
        *([pltpu.VMEM((_CHUNK_IDX, 32), jnp.float32)] * _RING),
        pltpu.VMEM((_CHUNK_IDX, _EMB), jnp.float32),
        pltpu.VMEM((_CHUNK_IDX, _EMB), jnp.float32),
        pltpu.VMEM((_SEG_PER_W, 32), jnp.float32),
        pltpu.VMEM((_QSEG_PER_W, _EMB), jnp.float32),
        *([pltpu.SemaphoreType.DMA] * (_RING + 2)),
    ],
  )(_sc_body)


def _tc_body(outs_ref, outq_ref, mb_ref, wout_ref, wfin_ref, o_ref, hf_scr):
    j = pl.program_id(0)

    @pl.when(j == 0)
    def _():
        mem = outs_ref[:, :, 0:16] + mb_ref[...][None, :, :]
        q = outq_ref[...]
        logits = jnp.sum(mem * q[:, None, :], axis=2)
        m = jnp.max(logits, axis=1, keepdims=True)
        e = jnp.exp(logits - m)
        probs = e / jnp.sum(e, axis=1, keepdims=True)
        outr = outs_ref[:, :, 16:32]
        lay = jnp.sum(outr * probs[:, :, None], axis=1)
        a = q + lay
        act = jnp.maximum(
            jnp.dot(a, wout_ref[...], preferred_element_type=jnp.float32), 0.0)
        r = act.reshape(_B, _NI, _LOUT)
        hf = r[:, :_N, :] + r[:, _NI - 1:_NI, :]
        hf_scr[...] = hf.reshape(_B * _N, _LOUT)

    o_ref[...] = jnp.dot(hf_scr[...], wfin_ref[...],
                         preferred_element_type=jnp.float32)


_tc_call = pl.pallas_call(
    _tc_body,
    grid=(_NVB,),
    in_specs=[
        pl.BlockSpec((_NQSEG, _MEM, 32), lambda j: (0, 0, 0)),
        pl.BlockSpec((_NQSEG, _EMB), lambda j: (0, 0)),
        pl.BlockSpec((_MEM, _EMB), lambda j: (0, 0)),
        pl.BlockSpec((_EMB, _LOUT), lambda j: (0, 0)),
        pl.BlockSpec((_LOUT, _TV), lambda j: (0, j)),
    ],
    out_specs=pl.BlockSpec((_B * _N, _TV), lambda j: (0, j)),
    out_shape=jax.ShapeDtypeStruct((_B * _N, _VOCAB), jnp.float32),
    scratch_shapes=[pltpu.VMEM((_B * _N, _LOUT), jnp.float32)],
)


def kernel(node_fts, edge_fts, graph_fts, adj_mat, hidden,
           query_biases, stories_biases, memory_contents, output_biases,
           w_output_linear, w_final):
    del hidden
    f32 = jnp.float32
    nil = jnp.zeros((1, _EMB), f32)
    tab = jnp.concatenate(
        [jnp.concatenate([stories_biases, nil], axis=0),
         jnp.concatenate([output_biases, nil], axis=0)], axis=1)
    qtab = jnp.concatenate([query_biases, nil], axis=0)

    ef = edge_fts * adj_mat[..., None]
    ef = jnp.pad(ef, ((0, 0), (0, 1), (0, 1), (0, 0)))
    idx_s = ef.astype(jnp.int32).reshape(_NW, _CHUNKS, _CHUNK_IDX)
    qf = jnp.concatenate([node_fts, graph_fts[:, None]], axis=1)
    idx_q = qf.astype(jnp.int32).reshape(_NW, _QCHUNKS, _CHUNK_IDX)

    enc = jnp.asarray(_ENC_NP)

    outs, outq = _make_sc_call()(tab, qtab, idx_s, idx_q, enc)
    outs = outs.reshape(_NQSEG, _MEM, 32)

    full = _tc_call(outs, outq, memory_contents, w_output_linear, w_final)
    return full.reshape(_B, _N, _VOCAB)


# speedup vs baseline: 7.7791x; 1.0885x over previous
import functools

import numpy as np
import jax
import jax.numpy as jnp
from jax import lax
from jax.experimental import pallas as pl
from jax.experimental.pallas import tpu as pltpu
from jax.experimental.pallas import tpu_sc as plsc

_B = 4
_N = 63
_SENT = 32
_EMB = 16
_MEM = 64
_LOUT = 128
_VOCAB = 100000

_NI = _N + 1
_NSEG = _B * _NI * _MEM
_NQSEG = _B * _NI

_NW = 32
_SEG_PER_W = _NSEG // _NW
_CHUNK_SEGS = 4
_CHUNK_IDX = _CHUNK_SEGS * _SENT
_CHUNKS = _SEG_PER_W // _CHUNK_SEGS
_QSEG_PER_W = _NQSEG // _NW
_QCHUNKS = _QSEG_PER_W // _CHUNK_SEGS

_TV = 4096
_NVB = 25


def _pos_encoding(sentence_size, embedding_size):
    encoding = np.ones((embedding_size, sentence_size), dtype=np.float32)
    ls = sentence_size + 1
    le = embedding_size + 1
    for i in range(1, le):
        for j in range(1, ls):
            encoding[i - 1, j - 1] = (i - (le - 1) / 2) * (j - (ls - 1) / 2)
    encoding = 1 + 4 * encoding / embedding_size / sentence_size
    return np.transpose(encoding)


_ENC_NP = _pos_encoding(_SENT, _EMB)


_RING = 8


def _sc_body(tab_hbm, qtab_hbm, idxs_hbm, idxq_hbm, enc_hbm,
             outs_hbm, outq_hbm,
             idx_v, qidx_v, enc_v, *rest):
    bufs = list(rest[:_RING])
    qrows_a, qrows_b, out_v, qout_v = rest[_RING:_RING + 4]
    sems = list(rest[_RING + 4:2 * _RING + 4])
    qga, qgb = rest[2 * _RING + 4:]
    wid = lax.axis_index("c") * 16 + lax.axis_index("s")

    pltpu.sync_copy(idxs_hbm.at[wid], idx_v)
    pltpu.sync_copy(idxq_hbm.at[wid], qidx_v)
    pltpu.sync_copy(enc_hbm, enc_v)

    pltpu.async_copy(qtab_hbm.at[qidx_v.at[0]], qrows_a, qga)
    pltpu.async_copy(qtab_hbm.at[qidx_v.at[1]], qrows_b, qgb)

    def fire(c, rows_ref, sem):
        pltpu.async_copy(tab_hbm.at[idx_v.at[c]], rows_ref, sem)

    def wait(rows_ref, sem):
        pltpu.make_async_copy(tab_hbm.at[idx_v.at[0]], rows_ref, sem).wait()

    def compute(c, rows_ref):
        am = [None] * _CHUNK_SEGS
        ao = [None] * _CHUNK_SEGS
        for s in range(_SENT):
            e = enc_v[s]
            for k in range(_CHUNK_SEGS):
                row = rows_ref[k * _SENT + s, :]
                sr, orow = plsc.unpack(row, format=plsc.PackFormat.INTERLEAVED)
                rm = sr * e
                ro = orow * e
                am[k] = rm if s == 0 else am[k] + rm
                ao[k] = ro if s == 0 else ao[k] + ro
        for k in range(_CHUNK_SEGS):
            seg = c * _CHUNK_SEGS + k
            out_v[seg, 0:16] = am[k]
            out_v[seg, 16:32] = ao[k]

    for r in range(_RING):
        fire(r, bufs[r], sems[r])

    def step(t, carry):
        for r in range(_RING):
            c = _RING * t + r
            wait(bufs[r], sems[r])
            compute(c, bufs[r])

            @pl.when(t < _CHUNKS // _RING - 1)
            def _():
                fire(c + _RING, bufs[r], sems[r])

        return carry

    lax.fori_loop(0, _CHUNKS // _RING, step, 0)

    def qcompute(qrows_ref, kbase):
        for k in range(_CHUNK_SEGS):
            acc0 = qrows_ref[k * _SENT + 0, :] * enc_v[0]
            acc1 = qrows_ref[k * _SENT + 1, :] * enc_v[1]
            for s in range(2, _SENT, 2):
                acc0 = acc0 + qrows_ref[k * _SENT + s, :] * enc_v[s]
                acc1 = acc1 + qrows_ref[k * _SENT + s + 1, :] * enc_v[s + 1]
            qout_v[kbase + k, :] = acc0 + acc1

    pltpu.make_async_copy(qtab_hbm.at[qidx_v.at[0]], qrows_a, qga).wait()
    qcompute(qrows_a, 0)
    pltpu.make_async_copy(qtab_hbm.at[qidx_v.at[1]], qrows_b, qgb).wait()
    qcompute(qrows_b, _CHUNK_SEGS)

    pltpu.sync_copy(out_v, outs_hbm.at[pl.ds(wid * _SEG_PER_W, _SEG_PER_W), :])
    pltpu.sync_copy(qout_v, outq_hbm.at[pl.ds(wid * _QSEG_PER_W, _QSEG_PER_W), :])


@functools.lru_cache(maxsize=None)
def _make_sc_call():
  return functools.partial(
    pl.kernel,
    out_type=[
        jax.ShapeDtypeStruct((_NSEG, 32), jnp.float32),
        jax.ShapeDtypeStruct((_NQSEG, _EMB), jnp.float32),
    ],
    mesh=plsc.VectorSubcoreMesh(core_axis_name="c", subcore_axis_name="s"),
    compiler_params=pltpu.CompilerParams(use_tc_tiling_on_sc=False,
                                         needs_layout_passes=False),
    scratch_types=[
        pltpu.VMEM((_CHUNKS, _CHUNK_IDX), jnp.int32),
        pltpu.VMEM((_QCHUNKS, _CHUNK_IDX), jnp.int32),
        pltpu.VMEM((_SENT, _EMB), jnp.float32),
        *([pltpu.VMEM((_CHUNK_IDX, 32), jnp.bfloat16)] * _RING),
        pltpu.VMEM((_CHUNK_IDX, _EMB), jnp.float32),
        pltpu.VMEM((_CHUNK_IDX, _EMB), jnp.float32),
        pltpu.VMEM((_SEG_PER_W, 32), jnp.float32),
        pltpu.VMEM((_QSEG_PER_W, _EMB), jnp.float32),
        *([pltpu.SemaphoreType.DMA] * (_RING + 2)),
    ],
  )(_sc_body)


def _tc_body(outs_ref, outq_ref, mb_ref, wout_ref, wfin_ref, o_ref, hf_scr):
    j = pl.program_id(0)

    @pl.when(j == 0)
    def _():
        mem = outs_ref[:, :, 0:16] + mb_ref[...][None, :, :]
        q = outq_ref[...]
        logits = jnp.sum(mem * q[:, None, :], axis=2)
        m = jnp.max(logits, axis=1, keepdims=True)
        e = jnp.exp(logits - m)
        probs = e / jnp.sum(e, axis=1, keepdims=True)
        outr = outs_ref[:, :, 16:32]
        lay = jnp.sum(outr * probs[:, :, None], axis=1)
        a = q + lay
        act = jnp.maximum(
            jnp.dot(a, wout_ref[...], preferred_element_type=jnp.float32), 0.0)
        r = act.reshape(_B, _NI, _LOUT)
        hf = r[:, :_N, :] + r[:, _NI - 1:_NI, :]
        hf_scr[...] = hf.reshape(_B * _N, _LOUT)

    o_ref[...] = jnp.dot(hf_scr[...], wfin_ref[...],
                         preferred_element_type=jnp.float32)


_tc_call = pl.pallas_call(
    _tc_body,
    grid=(_NVB,),
    in_specs=[
        pl.BlockSpec((_NQSEG, _MEM, 32), lambda j: (0, 0, 0)),
        pl.BlockSpec((_NQSEG, _EMB), lambda j: (0, 0)),
        pl.BlockSpec((_MEM, _EMB), lambda j: (0, 0)),
        pl.BlockSpec((_EMB, _LOUT), lambda j: (0, 0)),
        pl.BlockSpec((_LOUT, _TV), lambda j: (0, j)),
    ],
    out_specs=pl.BlockSpec((_B * _N, _TV), lambda j: (0, j)),
    out_shape=jax.ShapeDtypeStruct((_B * _N, _VOCAB), jnp.float32),
    scratch_shapes=[pltpu.VMEM((_B * _N, _LOUT), jnp.float32)],
)


def kernel(node_fts, edge_fts, graph_fts, adj_mat, hidden,
           query_biases, stories_biases, memory_contents, output_biases,
           w_output_linear, w_final):
    del hidden
    f32 = jnp.float32
    nil = jnp.zeros((1, _EMB), f32)
    tab = jnp.stack(
        [jnp.concatenate([stories_biases, nil], axis=0),
         jnp.concatenate([output_biases, nil], axis=0)],
        axis=2).reshape(_VOCAB, 2 * _EMB).astype(jnp.bfloat16)
    qtab = jnp.concatenate([query_biases, nil], axis=0)

    ef = edge_fts * adj_mat[..., None]
    ef = jnp.pad(ef, ((0, 0), (0, 1), (0, 1), (0, 0)))
    idx_s = ef.astype(jnp.int32).reshape(_NW, _CHUNKS, _CHUNK_IDX)
    qf = jnp.concatenate([node_fts, graph_fts[:, None]], axis=1)
    idx_q = qf.astype(jnp.int32).reshape(_NW, _QCHUNKS, _CHUNK_IDX)

    enc = jnp.asarray(_ENC_NP)

    outs, outq = _make_sc_call()(tab, qtab, idx_s, idx_q, enc)
    outs = outs.reshape(_NQSEG, _MEM, 32)

    full = _tc_call(outs, outq, memory_contents, w_output_linear, w_final)
    return full.reshape(_B, _N, _VOCAB)


# speedup vs baseline: 7.8042x; 1.0032x over previous
import functools

import numpy as np
import jax
import jax.numpy as jnp
from jax import lax
from jax.experimental import pallas as pl
from jax.experimental.pallas import tpu as pltpu
from jax.experimental.pallas import tpu_sc as plsc

_B = 4
_N = 63
_SENT = 32
_EMB = 16
_MEM = 64
_LOUT = 128
_VOCAB = 100000

_NI = _N + 1
_NSEG = _B * _NI * _MEM
_NQSEG = _B * _NI

_NW = 32
_SEG_PER_W = _NSEG // _NW
_CHUNK_SEGS = 4
_CHUNK_IDX = _CHUNK_SEGS * _SENT
_CHUNKS = _SEG_PER_W // _CHUNK_SEGS
_QSEG_PER_W = _NQSEG // _NW
_QCHUNKS = _QSEG_PER_W // _CHUNK_SEGS

_TV = 4096
_NVB = 25


def _pos_encoding(sentence_size, embedding_size):
    encoding = np.ones((embedding_size, sentence_size), dtype=np.float32)
    ls = sentence_size + 1
    le = embedding_size + 1
    for i in range(1, le):
        for j in range(1, ls):
            encoding[i - 1, j - 1] = (i - (le - 1) / 2) * (j - (ls - 1) / 2)
    encoding = 1 + 4 * encoding / embedding_size / sentence_size
    return np.transpose(encoding)


_ENC_NP = _pos_encoding(_SENT, _EMB)


_RING = 8


def _sc_body(tab_hbm, qtab_hbm, idxs_hbm, idxq_hbm, enc_hbm,
             outs_hbm, outq_hbm,
             idx_v, qidx_v, enc_v, *rest):
    bufs = list(rest[:_RING])
    qrows_a, qrows_b, out_v, qout_v = rest[_RING:_RING + 4]
    sems = list(rest[_RING + 4:2 * _RING + 4])
    qga, qgb = rest[2 * _RING + 4:]
    wid = lax.axis_index("c") * 16 + lax.axis_index("s")

    pltpu.sync_copy(idxs_hbm.at[wid], idx_v)
    pltpu.sync_copy(idxq_hbm.at[wid], qidx_v)
    pltpu.sync_copy(enc_hbm, enc_v)

    pltpu.async_copy(qtab_hbm.at[qidx_v.at[0]], qrows_a, qga)
    pltpu.async_copy(qtab_hbm.at[qidx_v.at[1]], qrows_b, qgb)

    def fire(c, rows_ref, sem):
        pltpu.async_copy(tab_hbm.at[idx_v.at[c]], rows_ref, sem)

    def wait(rows_ref, sem):
        pltpu.make_async_copy(tab_hbm.at[idx_v.at[0]], rows_ref, sem).wait()

    def compute(c, rows_ref):
        am = [None] * _CHUNK_SEGS
        ao = [None] * _CHUNK_SEGS
        for s in range(_SENT):
            e = enc_v[s]
            for k in range(_CHUNK_SEGS):
                row = rows_ref[k * _SENT + s, :]
                sr, orow = plsc.unpack(row, format=plsc.PackFormat.INTERLEAVED)
                rm = sr * e
                ro = orow * e
                am[k] = rm if s == 0 else am[k] + rm
                ao[k] = ro if s == 0 else ao[k] + ro
        for k in range(_CHUNK_SEGS):
            seg = c * _CHUNK_SEGS + k
            out_v[seg, 0:16] = am[k]
            out_v[seg, 16:32] = ao[k]

    for r in range(_RING):
        fire(r, bufs[r], sems[r])

    def step(t, carry):
        for r in range(_RING):
            c = _RING * t + r
            wait(bufs[r], sems[r])
            compute(c, bufs[r])

            @pl.when(t < _CHUNKS // _RING - 1)
            def _():
                fire(c + _RING, bufs[r], sems[r])

        return carry

    lax.fori_loop(0, _CHUNKS // _RING, step, 0)

    def qcompute(qrows_ref, kbase):
        for k in range(_CHUNK_SEGS):
            acc0 = qrows_ref[k * _SENT + 0, :] * enc_v[0]
            acc1 = qrows_ref[k * _SENT + 1, :] * enc_v[1]
            for s in range(2, _SENT, 2):
                acc0 = acc0 + qrows_ref[k * _SENT + s, :] * enc_v[s]
                acc1 = acc1 + qrows_ref[k * _SENT + s + 1, :] * enc_v[s + 1]
            qout_v[kbase + k, :] = acc0 + acc1

    pltpu.make_async_copy(qtab_hbm.at[qidx_v.at[0]], qrows_a, qga).wait()
    qcompute(qrows_a, 0)
    pltpu.make_async_copy(qtab_hbm.at[qidx_v.at[1]], qrows_b, qgb).wait()
    qcompute(qrows_b, _CHUNK_SEGS)

    pltpu.sync_copy(out_v, outs_hbm.at[pl.ds(wid * _SEG_PER_W, _SEG_PER_W), :])
    pltpu.sync_copy(qout_v, outq_hbm.at[pl.ds(wid * _QSEG_PER_W, _QSEG_PER_W), :])


@functools.lru_cache(maxsize=None)
def _make_sc_call():
  return functools.partial(
    pl.kernel,
    out_type=[
        jax.ShapeDtypeStruct((_NSEG, 32), jnp.float32),
        jax.ShapeDtypeStruct((_NQSEG, _EMB), jnp.float32),
    ],
    mesh=plsc.VectorSubcoreMesh(core_axis_name="c", subcore_axis_name="s"),
    compiler_params=pltpu.CompilerParams(use_tc_tiling_on_sc=False,
                                         needs_layout_passes=False),
    scratch_types=[
        pltpu.VMEM((_CHUNKS, _CHUNK_IDX), jnp.int32),
        pltpu.VMEM((_QCHUNKS, _CHUNK_IDX), jnp.int32),
        pltpu.VMEM((_SENT, _EMB), jnp.float32),
        *([pltpu.VMEM((_CHUNK_IDX, 32), jnp.bfloat16)] * _RING),
        pltpu.VMEM((_CHUNK_IDX, _EMB), jnp.float32),
        pltpu.VMEM((_CHUNK_IDX, _EMB), jnp.float32),
        pltpu.VMEM((_SEG_PER_W, 32), jnp.float32),
        pltpu.VMEM((_QSEG_PER_W, _EMB), jnp.float32),
        *([pltpu.SemaphoreType.DMA] * (_RING + 2)),
    ],
  )(_sc_body)


def _tc_body(outs_ref, outq_ref, mb_ref, wout_ref, wfin_ref, o_ref, hf_scr):
    j = pl.program_id(0)
    b = pl.program_id(1)

    @pl.when((j == 0) & (b == 0))
    def _():
        mem = outs_ref[:, :, 0:16] + mb_ref[...][None, :, :]
        q = outq_ref[...]
        logits = jnp.sum(mem * q[:, None, :], axis=2)
        m = jnp.max(logits, axis=1, keepdims=True)
        e = jnp.exp(logits - m)
        probs = e / jnp.sum(e, axis=1, keepdims=True)
        outr = outs_ref[:, :, 16:32]
        lay = jnp.sum(outr * probs[:, :, None], axis=1)
        a = q + lay
        act = jnp.maximum(
            jnp.dot(a, wout_ref[...], preferred_element_type=jnp.float32), 0.0)
        r = act.reshape(_B, _NI, _LOUT)
        hf = r[:, :_N, :] + r[:, _NI - 1:_NI, :]
        hf_scr[...] = hf.reshape(_B * _N, _LOUT)

    o_ref[0] = jnp.dot(hf_scr[pl.ds(b * _N, _N), :], wfin_ref[...],
                       preferred_element_type=jnp.float32)


_tc_call = pl.pallas_call(
    _tc_body,
    grid=(_NVB, _B),
    in_specs=[
        pl.BlockSpec((_NQSEG, _MEM, 32), lambda j, b: (0, 0, 0)),
        pl.BlockSpec((_NQSEG, _EMB), lambda j, b: (0, 0)),
        pl.BlockSpec((_MEM, _EMB), lambda j, b: (0, 0)),
        pl.BlockSpec((_EMB, _LOUT), lambda j, b: (0, 0)),
        pl.BlockSpec((_LOUT, _TV), lambda j, b: (0, j)),
    ],
    out_specs=pl.BlockSpec((1, _N, _TV), lambda j, b: (b, 0, j)),
    out_shape=jax.ShapeDtypeStruct((_B, _N, _VOCAB), jnp.float32),
    scratch_shapes=[pltpu.VMEM((_B * _N, _LOUT), jnp.float32)],
)


def kernel(node_fts, edge_fts, graph_fts, adj_mat, hidden,
           query_biases, stories_biases, memory_contents, output_biases,
           w_output_linear, w_final):
    del hidden
    f32 = jnp.float32
    nil = jnp.zeros((1, _EMB), f32)
    tab = jnp.stack(
        [jnp.concatenate([stories_biases, nil], axis=0),
         jnp.concatenate([output_biases, nil], axis=0)],
        axis=2).reshape(_VOCAB, 2 * _EMB).astype(jnp.bfloat16)
    qtab = jnp.concatenate([query_biases, nil], axis=0)

    ef = edge_fts * adj_mat[..., None]
    ef = jnp.pad(ef, ((0, 0), (0, 1), (0, 1), (0, 0)))
    idx_s = ef.astype(jnp.int32).reshape(_NW, _CHUNKS, _CHUNK_IDX)
    qf = jnp.concatenate([node_fts, graph_fts[:, None]], axis=1)
    idx_q = qf.astype(jnp.int32).reshape(_NW, _QCHUNKS, _CHUNK_IDX)

    enc = jnp.asarray(_ENC_NP)

    outs, outq = _make_sc_call()(tab, qtab, idx_s, idx_q, enc)
    outs = outs.reshape(_NQSEG, _MEM, 32)

    return _tc_call(outs, outq, memory_contents, w_output_linear, w_final)
